# baseline (device time: 17143 ns/iter reference)
import jax
import jax.numpy as jnp
from jax import lax
from jax.experimental import pallas as pl
from jax.experimental.pallas import tpu as pltpu

N_Z = 4
T = 256
D = 512
V_LOCAL = 4096
G = 8
BLK = V_LOCAL // G


def kernel(x, W, labels):
    labels2 = labels.reshape(T, 1)

    def body(x_ref, w_ref, lbl_ref, out_ref, comm_ref, send_sems, recv_sems):
        my_x = lax.axis_index("x")
        my_y = lax.axis_index("y")
        my_z = lax.axis_index("z")
        g = pl.program_id(0)

        @pl.when(g == 0)
        def _():
            barrier = pltpu.get_barrier_semaphore()
            for k in range(1, N_Z):
                peer = lax.rem(my_z + k, N_Z)
                pl.semaphore_signal(
                    barrier, inc=1,
                    device_id=(my_x, my_y, peer),
                    device_id_type=pl.DeviceIdType.MESH,
                )
            comm_ref[my_z, 0, :] = jnp.zeros((T,), jnp.float32)
            comm_ref[my_z, 1, :] = jnp.zeros((T,), jnp.float32)

        xv = x_ref[...].astype(jnp.bfloat16)
        wv = w_ref[...].astype(jnp.bfloat16)
        logits = jnp.dot(xv, wv, preferred_element_type=jnp.float32)

        s_blk = jnp.sum(jnp.exp(logits), axis=1)
        col = lax.broadcasted_iota(jnp.int32, (T, BLK), 1)
        idx = lbl_ref[...] - (my_z * V_LOCAL + g * BLK)
        t_blk = jnp.sum(jnp.where(col == idx, logits, 0.0), axis=1)

        comm_ref[my_z, 0, :] += s_blk
        comm_ref[my_z, 1, :] += t_blk

        @pl.when(g == G - 1)
        def _():
            barrier = pltpu.get_barrier_semaphore()
            pl.semaphore_wait(barrier, N_Z - 1)

            sends = []
            for k in range(1, N_Z):
                peer = lax.rem(my_z + k, N_Z)
                rdma = pltpu.make_async_remote_copy(
                    src_ref=comm_ref.at[my_z],
                    dst_ref=comm_ref.at[my_z],
                    send_sem=send_sems.at[k - 1],
                    recv_sem=recv_sems.at[my_z],
                    device_id=(my_x, my_y, peer),
                    device_id_type=pl.DeviceIdType.MESH,
                )
                rdma.start()
                sends.append(rdma)

            for k in range(1, N_Z):
                origin = lax.rem(my_z - k + N_Z, N_Z)
                recv = pltpu.make_async_remote_copy(
                    src_ref=comm_ref.at[origin],
                    dst_ref=comm_ref.at[origin],
                    send_sem=send_sems.at[0],
                    recv_sem=recv_sems.at[origin],
                    device_id=(my_x, my_y, origin),
                    device_id_type=pl.DeviceIdType.MESH,
                )
                recv.wait_recv()

            s_g = comm_ref[0, 0, :]
            t_g = comm_ref[0, 1, :]
            for k in range(1, N_Z):
                s_g = s_g + comm_ref[k, 0, :]
                t_g = t_g + comm_ref[k, 1, :]
            out_ref[...] = jnp.log(s_g) - t_g

            for rdma in sends:
                rdma.wait_send()

    return pl.pallas_call(
        body,
        grid=(G,),
        out_shape=jax.ShapeDtypeStruct((T,), jnp.float32),
        in_specs=[
            pl.BlockSpec((T, D), lambda g: (0, 0), memory_space=pltpu.VMEM),
            pl.BlockSpec((D, BLK), lambda g: (0, g), memory_space=pltpu.VMEM),
            pl.BlockSpec((T, 1), lambda g: (0, 0), memory_space=pltpu.VMEM),
        ],
        out_specs=pl.BlockSpec((T,), lambda g: (0,), memory_space=pltpu.VMEM),
        scratch_shapes=[
            pltpu.VMEM((N_Z, 2, T), jnp.float32),
            pltpu.SemaphoreType.DMA((N_Z - 1,)),
            pltpu.SemaphoreType.DMA((N_Z,)),
        ],
        compiler_params=pltpu.CompilerParams(
            collective_id=0,
            dimension_semantics=("arbitrary",),
        ),
    )(x, W, labels2)


# device time: 13116 ns/iter; 1.3070x vs baseline; 1.3070x over previous
import jax
import jax.numpy as jnp
from jax import lax
from jax.experimental import pallas as pl
from jax.experimental.pallas import tpu as pltpu

N_Z = 4
T = 256
V_LOCAL = 4096


def kernel(x, W, labels):
    labels2 = labels.reshape(T, 1)

    def body(x_ref, w_ref, lbl_ref, out_ref, comm_ref, send_sems, recv_sems):
        my_x = lax.axis_index("x")
        my_y = lax.axis_index("y")
        my_z = lax.axis_index("z")

        barrier = pltpu.get_barrier_semaphore()
        for k in range(1, N_Z):
            peer = lax.rem(my_z + k, N_Z)
            pl.semaphore_signal(
                barrier, inc=1,
                device_id=(my_x, my_y, peer),
                device_id_type=pl.DeviceIdType.MESH,
            )

        xv = x_ref[...].astype(jnp.bfloat16)
        wv = w_ref[...].astype(jnp.bfloat16)
        logits = jnp.dot(xv, wv, preferred_element_type=jnp.float32)

        s = jnp.sum(jnp.exp(logits), axis=1)
        col = lax.broadcasted_iota(jnp.int32, (T, V_LOCAL), 1)
        idx = lbl_ref[...] - my_z * V_LOCAL
        t = jnp.sum(jnp.where(col == idx, logits, 0.0), axis=1)

        comm_ref[my_z, 0, :] = s
        comm_ref[my_z, 1, :] = t

        pl.semaphore_wait(barrier, N_Z - 1)
        sends = []
        for k in range(1, N_Z):
            peer = lax.rem(my_z + k, N_Z)
            rdma = pltpu.make_async_remote_copy(
                src_ref=comm_ref.at[my_z],
                dst_ref=comm_ref.at[my_z],
                send_sem=send_sems.at[k - 1],
                recv_sem=recv_sems.at[my_z],
                device_id=(my_x, my_y, peer),
                device_id_type=pl.DeviceIdType.MESH,
            )
            rdma.start()
            sends.append(rdma)

        for k in range(1, N_Z):
            origin = lax.rem(my_z - k + N_Z, N_Z)
            recv = pltpu.make_async_remote_copy(
                src_ref=comm_ref.at[origin],
                dst_ref=comm_ref.at[origin],
                send_sem=send_sems.at[0],
                recv_sem=recv_sems.at[origin],
                device_id=(my_x, my_y, origin),
                device_id_type=pl.DeviceIdType.MESH,
            )
            recv.wait_recv()

        s_g = comm_ref[0, 0, :]
        t_g = comm_ref[0, 1, :]
        for k in range(1, N_Z):
            s_g = s_g + comm_ref[k, 0, :]
            t_g = t_g + comm_ref[k, 1, :]
        out_ref[...] = jnp.log(s_g) - t_g

        for rdma in sends:
            rdma.wait_send()

    return pl.pallas_call(
        body,
        out_shape=jax.ShapeDtypeStruct((T,), jnp.float32),
        in_specs=[
            pl.BlockSpec(memory_space=pltpu.VMEM),
            pl.BlockSpec(memory_space=pltpu.VMEM),
            pl.BlockSpec(memory_space=pltpu.VMEM),
        ],
        out_specs=pl.BlockSpec(memory_space=pltpu.VMEM),
        scratch_shapes=[
            pltpu.VMEM((N_Z, 2, T), jnp.float32),
            pltpu.SemaphoreType.DMA((N_Z - 1,)),
            pltpu.SemaphoreType.DMA((N_Z,)),
        ],
        compiler_params=pltpu.CompilerParams(collective_id=0),
    )(x, W, labels2)
